# Initial kernel scaffold; baseline (speedup 1.0000x reference)
import jax
import jax.numpy as jnp
from jax.experimental import pallas as pl


def kernel(token_ids, embeddings):
    raise NotImplementedError("write your pallas kernel here")


# speedup vs baseline: 1.1032x; 1.1032x over previous
import functools

import jax
import jax.numpy as jnp
from jax import lax
from jax.experimental import pallas as pl
from jax.experimental.pallas import tpu as pltpu
from jax.experimental.pallas import tpu_sc as plsc

_NUM_CORES = 2
_NUM_SUBCORES = 16
_NUM_WORKERS = _NUM_CORES * _NUM_SUBCORES


@functools.lru_cache(maxsize=None)
def _build(B: int, V: int, D: int):
  assert B % _NUM_WORKERS == 0
  b_per_w = B // _NUM_WORKERS
  chunk = b_per_w
  n_chunks = 1
  while chunk * (D + 1) * 4 > 400_000 or chunk % 8:
    n_chunks *= 2
    chunk = b_per_w // n_chunks
  assert chunk * n_chunks == b_per_w and chunk % 8 == 0

  mesh = plsc.VectorSubcoreMesh(
      core_axis_name="c", subcore_axis_name="s",
      num_cores=_NUM_CORES, num_subcores=_NUM_SUBCORES)

  @functools.partial(
      pl.kernel,
      out_type=jax.ShapeDtypeStruct((B, D), jnp.float32),
      mesh=mesh,
      scratch_types=[
          pltpu.VMEM((chunk,), jnp.int32),
          pltpu.VMEM((chunk, D), jnp.float32),
          pltpu.SemaphoreType.DMA,
      ],
      compiler_params=pltpu.CompilerParams(use_tc_tiling_on_sc=False),
  )
  def gather_kernel(tok_hbm, tab_hbm, out_hbm, idx_v, rows_v, sem):
    wid = lax.axis_index("s") * _NUM_CORES + lax.axis_index("c")
    base = wid * b_per_w

    @pl.loop(0, n_chunks)
    def _chunk(g):
      off = base + g * chunk
      pltpu.sync_copy(tok_hbm.at[pl.ds(off, chunk)], idx_v)
      pltpu.async_copy(tab_hbm.at[idx_v], rows_v, sem).wait()
      pltpu.sync_copy(rows_v, out_hbm.at[pl.ds(off, chunk)])

  return gather_kernel


def kernel(token_ids, embeddings):
  orig_shape = token_ids.shape
  flat = jnp.reshape(token_ids.astype(jnp.int32), (-1,))
  B = flat.shape[0]
  V, D = embeddings.shape
  out = _build(B, V, D)(flat, embeddings)
  return jnp.reshape(out, (*orig_shape, D))


# speedup vs baseline: 1.2146x; 1.1010x over previous
import functools

import jax
import jax.numpy as jnp
from jax import lax
from jax.experimental import pallas as pl
from jax.experimental.pallas import tpu as pltpu
from jax.experimental.pallas import tpu_sc as plsc

_NUM_CORES = 2
_NUM_SUBCORES = 16
_NUM_WORKERS = _NUM_CORES * _NUM_SUBCORES


@functools.lru_cache(maxsize=None)
def _build(T: int, S: int, V: int, D: int):
  assert D % 8 == 0 and T % (128 * _NUM_WORKERS) == 0
  CC = D // 8
  NT = T // 128
  JPW = NT // _NUM_WORKERS
  CH = 128 * JPW

  mesh = plsc.VectorSubcoreMesh(
      core_axis_name="c", subcore_axis_name="s",
      num_cores=_NUM_CORES, num_subcores=_NUM_SUBCORES)

  @functools.partial(
      pl.kernel,
      out_type=jax.ShapeDtypeStruct((S, CC, NT, 8, 128), jnp.float32),
      mesh=mesh,
      scratch_types=[
          [pltpu.VMEM((CH,), jnp.int32)] * 2,
          [pltpu.VMEM((CH, D), jnp.float32)] * 2,
          [pltpu.VMEM((JPW, CC, 8, 128), jnp.float32)] * 2,
          [pltpu.SemaphoreType.DMA] * 2,
          [pltpu.SemaphoreType.DMA] * 2,
          [pltpu.SemaphoreType.DMA] * 2,
      ],
      compiler_params=pltpu.CompilerParams(
          use_tc_tiling_on_sc=False, needs_layout_passes=False),
  )
  def gather_kernel(tok_hbm, tab_hbm, out_hbm, idx_v, rows_v, trans_v,
                    isem, gsem, osem):
    w = lax.axis_index("s") * _NUM_CORES + lax.axis_index("c")
    iota16 = lax.iota(jnp.int32, 16)

    def idx_copy(s, b):
      return pltpu.make_async_copy(
          tok_hbm.at[pl.ds(s * T + w * CH, CH)], idx_v[b], isem[b])

    def gather(b):
      return pltpu.make_async_copy(
          tab_hbm.at[idx_v[b]], rows_v[b], gsem[b])

    def out_copies(s, b):
      return [pltpu.make_async_copy(
                  trans_v[b].at[j, cc],
                  out_hbm.at[s, cc, JPW * w + j], osem[b])
              for j in range(JPW) for cc in range(CC)]

    idx_copy(0, 0).start()
    idx_copy(0, 0).wait()
    gather(0).start()
    idx_copy(1, 1).start()

    def _body(s, b):
      gather(b).wait()

      @pl.when(s + 2 < S)
      def _():
        idx_copy(s + 2, b).start()

      @pl.when(s >= 2)
      def _():
        for c in out_copies(s - 2, b):
          c.wait()

      @pl.loop(0, D)
      def _ch(c):
        cc = c // 8
        sub = lax.rem(c, 8)
        colv = jnp.full((16,), 0, jnp.int32) + c
        for j in range(JPW):
          for k in range(8):
            v = plsc.load_gather(
                rows_v[b], [iota16 + (j * 128 + 16 * k), colv])
            trans_v[b][j, cc, sub, pl.ds(16 * k, 16)] = v

      for cpy in out_copies(s, b):
        cpy.start()

      @pl.when(s + 1 < S)
      def _():
        idx_copy(s + 1, 1 - b).wait()
        gather(1 - b).start()

    @pl.loop(0, S)
    def _chunk(s):
      b = lax.rem(s, 2)

      @pl.when(b == 0)
      def _():
        _body(s, 0)

      @pl.when(b == 1)
      def _():
        _body(s, 1)

    for c in out_copies(S - 2, (S - 2) % 2):
      c.wait()
    for c in out_copies(S - 1, (S - 1) % 2):
      c.wait()

  return gather_kernel


def kernel(token_ids, embeddings):
  T, S = token_ids.shape
  V, D = embeddings.shape
  tok_sm = jnp.reshape(jnp.transpose(token_ids.astype(jnp.int32)), (-1,))
  zero = lax.optimization_barrier(jnp.float32(0.0))
  out5 = _build(T, S, V, D)(tok_sm, embeddings + zero)
  out = jnp.reshape(jnp.transpose(out5, (2, 4, 0, 1, 3)), (T, S, D))
  return out


# speedup vs baseline: 1.3141x; 1.0819x over previous
import functools

import jax
import jax.numpy as jnp
from jax import lax
from jax.experimental import pallas as pl
from jax.experimental.pallas import tpu as pltpu
from jax.experimental.pallas import tpu_sc as plsc

_NUM_CORES = 2
_NUM_SUBCORES = 16
_NUM_WORKERS = _NUM_CORES * _NUM_SUBCORES


@functools.lru_cache(maxsize=None)
def _build(T: int, S: int, V: int, D: int):
  assert D == 32 and T % (128 * _NUM_WORKERS) == 0
  CC = D // 8
  NT = T // 128
  JPW = NT // _NUM_WORKERS
  CH = 128 * JPW

  mesh = plsc.VectorSubcoreMesh(
      core_axis_name="c", subcore_axis_name="s",
      num_cores=_NUM_CORES, num_subcores=_NUM_SUBCORES)

  @functools.partial(
      pl.kernel,
      out_type=jax.ShapeDtypeStruct((S * CC * NT * 1024,), jnp.float32),
      mesh=mesh,
      scratch_types=[
          [pltpu.VMEM((CH,), jnp.int32)] * 2,
          [pltpu.VMEM((CH, D), jnp.float32)] * 2,
          [pltpu.VMEM((JPW * CC * 1024,), jnp.float32)] * 2,
          [pltpu.SemaphoreType.DMA] * 2,
          [pltpu.SemaphoreType.DMA] * 2,
          [pltpu.SemaphoreType.DMA] * 2,
      ],
      compiler_params=pltpu.CompilerParams(
          use_tc_tiling_on_sc=False, needs_layout_passes=False),
  )
  def gather_kernel(tok_hbm, tab_hbm, out_hbm, idx_v, rows_v, trans_v,
                    isem, gsem, osem):
    w = lax.axis_index("s") * _NUM_CORES + lax.axis_index("c")
    iota16 = lax.iota(jnp.int32, 16)
    idx_lo = (iota16 // 8) * 1024 + (iota16 % 8) * 128
    idx_hi = idx_lo + (16 // 8) * 1024

    def idx_copy(s, b):
      return pltpu.make_async_copy(
          tok_hbm.at[pl.ds(s * T + w * CH, CH)], idx_v[b], isem[b])

    def gather(b):
      return pltpu.make_async_copy(
          tab_hbm.at[idx_v[b]], rows_v[b], gsem[b])

    def out_copies(s, b):
      return [pltpu.make_async_copy(
                  trans_v[b].at[pl.ds((j * CC + cc) * 1024, 1024)],
                  out_hbm.at[pl.ds(
                      ((s * CC + cc) * NT + JPW * w + j) * 1024, 1024)],
                  osem[b])
              for j in range(JPW) for cc in range(CC)]

    idx_copy(0, 0).start()
    idx_copy(0, 0).wait()
    gather(0).start()
    idx_copy(1, 1).start()

    def _body(s, b):
      gather(b).wait()

      @pl.when(s + 2 < S)
      def _():
        idx_copy(s + 2, b).start()

      @pl.when(s >= 2)
      def _():
        for c in out_copies(s - 2, b):
          c.wait()

      @pl.loop(0, 128, unroll=8)
      def _tok(l):
        for j in range(JPW):
          r = j * 128 + l
          base = j * (CC * 1024) + l
          v_lo = rows_v[b][r, pl.ds(0, 16)]
          plsc.store_scatter(trans_v[b], [idx_lo + base], v_lo)
          v_hi = rows_v[b][r, pl.ds(16, 16)]
          plsc.store_scatter(trans_v[b], [idx_hi + base], v_hi)

      for cpy in out_copies(s, b):
        cpy.start()

      @pl.when(s + 1 < S)
      def _():
        idx_copy(s + 1, 1 - b).wait()
        gather(1 - b).start()

    @pl.loop(0, S)
    def _chunk(s):
      b = lax.rem(s, 2)

      @pl.when(b == 0)
      def _():
        _body(s, 0)

      @pl.when(b == 1)
      def _():
        _body(s, 1)

    for c in out_copies(S - 2, (S - 2) % 2):
      c.wait()
    for c in out_copies(S - 1, (S - 1) % 2):
      c.wait()

  return gather_kernel


def kernel(token_ids, embeddings):
  T, S = token_ids.shape
  V, D = embeddings.shape
  tok_sm = jnp.reshape(jnp.transpose(token_ids.astype(jnp.int32)), (-1,))
  zero = lax.optimization_barrier(jnp.float32(0.0))
  out_flat = _build(T, S, V, D)(tok_sm, embeddings + zero)
  out5 = jnp.reshape(out_flat, (S, D // 8, T // 128, 8, 128))
  out = jnp.reshape(jnp.transpose(out5, (2, 4, 0, 1, 3)), (T, S, D))
  return out


# speedup vs baseline: 1.8428x; 1.4023x over previous
import functools

import jax
import jax.numpy as jnp
from jax import lax
from jax.experimental import pallas as pl
from jax.experimental.pallas import tpu as pltpu
from jax.experimental.pallas import tpu_sc as plsc

_NUM_CORES = 2
_NUM_SUBCORES = 16
_NUM_WORKERS = _NUM_CORES * _NUM_SUBCORES


@functools.lru_cache(maxsize=None)
def _build(T: int, S: int, V: int, D: int):
  assert D == 32 and T % (128 * _NUM_WORKERS) == 0
  CC = D // 8
  NT = T // 128
  JPW = NT // _NUM_WORKERS
  CH = 128 * JPW

  mesh = plsc.VectorSubcoreMesh(
      core_axis_name="c", subcore_axis_name="s",
      num_cores=_NUM_CORES, num_subcores=_NUM_SUBCORES)

  @functools.partial(
      pl.kernel,
      out_type=jax.ShapeDtypeStruct((S * CC * NT * 1024,), jnp.float32),
      mesh=mesh,
      scratch_types=[
          [pltpu.VMEM((CH,), jnp.int32)] * 2,
          [pltpu.VMEM((CH, D), jnp.float32)] * 2,
          [pltpu.VMEM((JPW * CC * 1024,), jnp.float32)] * 2,
          [pltpu.SemaphoreType.DMA] * 2,
          [pltpu.SemaphoreType.DMA] * 2,
          [pltpu.SemaphoreType.DMA] * 2,
      ],
      compiler_params=pltpu.CompilerParams(
          use_tc_tiling_on_sc=False, needs_layout_passes=False),
  )
  def gather_kernel(tok_hbm, tab_hbm, out_hbm, idx_v, rows_v, trans_v,
                    isem, gsem, osem):
    w = lax.axis_index("s") * _NUM_CORES + lax.axis_index("c")
    iota16 = lax.iota(jnp.int32, 16)
    idx_lo = (iota16 // 8) * 1024 + (iota16 % 8) * 128
    idx_hi = idx_lo + (16 // 8) * 1024

    def idx_copy(s, b):
      return pltpu.make_async_copy(
          tok_hbm.at[pl.ds(s * T + w * CH, CH)], idx_v[b], isem[b])

    def gather(b):
      return pltpu.make_async_copy(
          tab_hbm.at[idx_v[b]], rows_v[b], gsem[b])

    def out_copies(s, b):
      return [pltpu.make_async_copy(
                  trans_v[b].at[pl.ds((j * CC + cc) * 1024, 1024)],
                  out_hbm.at[pl.ds(
                      ((s * CC + cc) * NT + JPW * w + j) * 1024, 1024)],
                  osem[b])
              for j in range(JPW) for cc in range(CC)]

    idx_copy(0, 0).start()
    idx_copy(0, 0).wait()
    gather(0).start()
    idx_copy(1, 1).start()

    def _body(s, b):
      gather(b).wait()

      @pl.when(s + 2 < S)
      def _():
        idx_copy(s + 2, b).start()

      @pl.when(s + 1 < S)
      def _():
        idx_copy(s + 1, 1 - b).wait()
        gather(1 - b).start()

      @pl.when(s >= 2)
      def _():
        for c in out_copies(s - 2, b):
          c.wait()

      @pl.loop(0, 128, unroll=8)
      def _tok(l):
        for j in range(JPW):
          r = j * 128 + l
          base = j * (CC * 1024) + l
          v_lo = rows_v[b][r, pl.ds(0, 16)]
          plsc.store_scatter(trans_v[b], [idx_lo + base], v_lo)
          v_hi = rows_v[b][r, pl.ds(16, 16)]
          plsc.store_scatter(trans_v[b], [idx_hi + base], v_hi)

      for cpy in out_copies(s, b):
        cpy.start()

    @pl.loop(0, S)
    def _chunk(s):
      b = lax.rem(s, 2)

      @pl.when(b == 0)
      def _():
        _body(s, 0)

      @pl.when(b == 1)
      def _():
        _body(s, 1)

    for c in out_copies(S - 2, (S - 2) % 2):
      c.wait()
    for c in out_copies(S - 1, (S - 1) % 2):
      c.wait()

  return gather_kernel


def kernel(token_ids, embeddings):
  T, S = token_ids.shape
  V, D = embeddings.shape
  tok_sm = jnp.reshape(jnp.transpose(token_ids.astype(jnp.int32)), (-1,))
  out_flat = _build(T, S, V, D)(tok_sm, embeddings)
  out5 = jnp.reshape(out_flat, (S, D // 8, T // 128, 8, 128))
  out = jnp.reshape(jnp.transpose(out5, (2, 4, 0, 1, 3)), (T, S, D))
  return out


# speedup vs baseline: 2.0669x; 1.1216x over previous
import functools

import jax
import jax.numpy as jnp
from jax import lax
from jax.experimental import pallas as pl
from jax.experimental.pallas import tpu as pltpu
from jax.experimental.pallas import tpu_sc as plsc

_NUM_CORES = 2
_NUM_SUBCORES = 16
_NUM_WORKERS = _NUM_CORES * _NUM_SUBCORES


@functools.lru_cache(maxsize=None)
def _build(T: int, S: int, V: int, D: int):
  assert D == 32 and T % (128 * _NUM_WORKERS) == 0
  CC = D // 8
  NT = T // 128
  JPW = NT // _NUM_WORKERS
  CH = 128 * JPW

  mesh = plsc.VectorSubcoreMesh(
      core_axis_name="c", subcore_axis_name="s",
      num_cores=_NUM_CORES, num_subcores=_NUM_SUBCORES)

  @functools.partial(
      pl.kernel,
      out_type=jax.ShapeDtypeStruct((S * CC * NT * 1024,), jnp.float32),
      mesh=mesh,
      scratch_types=[
          [pltpu.VMEM((CH,), jnp.int32)] * 2,
          [pltpu.VMEM((CH, D), jnp.float32)] * 2,
          [pltpu.VMEM((CH * (D + 1),), jnp.float32)] * 2,
          [pltpu.VMEM((JPW * CC * 1024,), jnp.float32)] * 2,
          [pltpu.SemaphoreType.DMA] * 2,
          [pltpu.SemaphoreType.DMA] * 2,
          [pltpu.SemaphoreType.DMA] * 2,
      ],
      compiler_params=pltpu.CompilerParams(
          use_tc_tiling_on_sc=False, needs_layout_passes=False),
  )
  def gather_kernel(tok_hbm, tab_hbm, out_hbm, idx_v, rows_v, rows33_v,
                    trans_v, isem, gsem, osem):
    w = lax.axis_index("s") * _NUM_CORES + lax.axis_index("c")
    iota16 = lax.iota(jnp.int32, 16)
    iota33 = iota16 * (D + 1)

    def idx_copy(s, b):
      return pltpu.make_async_copy(
          tok_hbm.at[pl.ds(s * T + w * CH, CH)], idx_v[b], isem[b])

    def gather(b):
      return pltpu.make_async_copy(
          tab_hbm.at[idx_v[b]], rows_v[b], gsem[b])

    def out_copies(s, b):
      return [pltpu.make_async_copy(
                  trans_v[b].at[pl.ds((j * CC + cc) * 1024, 1024)],
                  out_hbm.at[pl.ds(
                      ((s * CC + cc) * NT + JPW * w + j) * 1024, 1024)],
                  osem[b])
              for j in range(JPW) for cc in range(CC)]

    idx_copy(0, 0).start()
    idx_copy(0, 0).wait()
    gather(0).start()
    idx_copy(1, 1).start()

    def _body(s, b):
      gather(b).wait()

      @pl.when(s + 2 < S)
      def _():
        idx_copy(s + 2, b).start()

      @pl.when(s + 1 < S)
      def _():
        idx_copy(s + 1, 1 - b).wait()
        gather(1 - b).start()

      @pl.when(s >= 2)
      def _():
        for c in out_copies(s - 2, b):
          c.wait()

      @pl.loop(0, CH, unroll=8)
      def _pad(r):
        rows33_v[b][pl.ds(r * (D + 1), 16)] = rows_v[b][r, pl.ds(0, 16)]
        rows33_v[b][pl.ds(r * (D + 1) + 16, 16)] = (
            rows_v[b][r, pl.ds(16, 16)])

      @pl.loop(0, D, unroll=2)
      def _chan(c):
        g = (c // 8) * 1024 + lax.rem(c, 8) * 128
        for j in range(JPW):
          jb = j * (CC * 1024)
          for k in range(8):
            src = iota33 + ((j * 128 + 16 * k) * (D + 1) + c)
            v = plsc.load_gather(rows33_v[b], [src])
            trans_v[b][pl.ds(jb + g + 16 * k, 16)] = v

      for cpy in out_copies(s, b):
        cpy.start()

    @pl.loop(0, S)
    def _chunk(s):
      b = lax.rem(s, 2)

      @pl.when(b == 0)
      def _():
        _body(s, 0)

      @pl.when(b == 1)
      def _():
        _body(s, 1)

    for c in out_copies(S - 2, (S - 2) % 2):
      c.wait()
    for c in out_copies(S - 1, (S - 1) % 2):
      c.wait()

  return gather_kernel


def kernel(token_ids, embeddings):
  T, S = token_ids.shape
  V, D = embeddings.shape
  tok_sm = jnp.reshape(jnp.transpose(token_ids.astype(jnp.int32)), (-1,))
  out_flat = _build(T, S, V, D)(tok_sm, embeddings)
  out5 = jnp.reshape(out_flat, (S, D // 8, T // 128, 8, 128))
  out = jnp.reshape(jnp.transpose(out5, (2, 4, 0, 1, 3)), (T, S, D))
  return out


# speedup vs baseline: 2.1386x; 1.0347x over previous
import functools

import jax
import jax.numpy as jnp
from jax import lax
from jax.experimental import pallas as pl
from jax.experimental.pallas import tpu as pltpu
from jax.experimental.pallas import tpu_sc as plsc

_NUM_CORES = 2
_NUM_SUBCORES = 16
_NUM_WORKERS = _NUM_CORES * _NUM_SUBCORES


@functools.lru_cache(maxsize=None)
def _build(T: int, S: int, V: int, D: int):
  assert D == 32 and T % (128 * _NUM_WORKERS) == 0
  CC = D // 8
  NT = T // 128
  JPW = NT // _NUM_WORKERS
  CH = 128 * JPW

  mesh = plsc.VectorSubcoreMesh(
      core_axis_name="c", subcore_axis_name="s",
      num_cores=_NUM_CORES, num_subcores=_NUM_SUBCORES)

  @functools.partial(
      pl.kernel,
      out_type=jax.ShapeDtypeStruct((S * CC * NT * 1024,), jnp.float32),
      mesh=mesh,
      scratch_types=[
          [pltpu.VMEM((CH,), jnp.int32)] * 2,
          [pltpu.VMEM((CH, D), jnp.float32)] * 2,
          [pltpu.VMEM((CH * (D + 1),), jnp.float32)] * 2,
          [pltpu.VMEM((JPW * CC * 1024,), jnp.float32)] * 2,
          [pltpu.SemaphoreType.DMA] * 2,
          [pltpu.SemaphoreType.DMA] * 2,
          [pltpu.SemaphoreType.DMA] * 2,
      ],
      compiler_params=pltpu.CompilerParams(
          use_tc_tiling_on_sc=False, needs_layout_passes=False),
  )
  def gather_kernel(tok_hbm, tab_hbm, out_hbm, idx_v, rows_v, rows33_v,
                    trans_v, isem, gsem, osem):
    w = lax.axis_index("s") * _NUM_CORES + lax.axis_index("c")
    iota16 = lax.iota(jnp.int32, 16)
    iota33 = iota16 * (D + 1)

    def idx_copy(s, b):
      return pltpu.make_async_copy(
          tok_hbm.at[pl.ds(s * T + w * CH, CH)], idx_v[b], isem[b])

    def gather(b):
      return pltpu.make_async_copy(
          tab_hbm.at[idx_v[b]], rows_v[b], gsem[b])

    def out_copies(s, b):
      return [pltpu.make_async_copy(
                  trans_v[b].at[pl.ds((j * CC + cc) * 1024, 1024)],
                  out_hbm.at[pl.ds(
                      ((s * CC + cc) * NT + JPW * w + j) * 1024, 1024)],
                  osem[b])
              for j in range(JPW) for cc in range(CC)]

    idx_copy(0, 0).start()
    idx_copy(0, 0).wait()
    gather(0).start()
    idx_copy(1, 1).start()

    def _body(s, b):
      gather(b).wait()

      @pl.when(s + 2 < S)
      def _():
        idx_copy(s + 2, b).start()

      @pl.when(s + 1 < S)
      def _():
        idx_copy(s + 1, 1 - b).wait()
        gather(1 - b).start()

      @pl.when(s >= 2)
      def _():
        for c in out_copies(s - 2, b):
          c.wait()

      @pl.loop(0, CH, unroll=16)
      def _pad(r):
        rows33_v[b][pl.ds(r * (D + 1), 16)] = rows_v[b][r, pl.ds(0, 16)]
        rows33_v[b][pl.ds(r * (D + 1) + 16, 16)] = (
            rows_v[b][r, pl.ds(16, 16)])

      @pl.loop(0, D, unroll=4)
      def _chan(c):
        g = (c // 8) * 1024 + lax.rem(c, 8) * 128
        for j in range(JPW):
          jb = j * (CC * 1024)
          for k in range(8):
            src = iota33 + ((j * 128 + 16 * k) * (D + 1) + c)
            v = plsc.load_gather(rows33_v[b], [src])
            trans_v[b][pl.ds(jb + g + 16 * k, 16)] = v

      for cpy in out_copies(s, b):
        cpy.start()

    @pl.loop(0, S)
    def _chunk(s):
      b = lax.rem(s, 2)

      @pl.when(b == 0)
      def _():
        _body(s, 0)

      @pl.when(b == 1)
      def _():
        _body(s, 1)

    for c in out_copies(S - 2, (S - 2) % 2):
      c.wait()
    for c in out_copies(S - 1, (S - 1) % 2):
      c.wait()

  return gather_kernel


def kernel(token_ids, embeddings):
  T, S = token_ids.shape
  V, D = embeddings.shape
  tok_sm = jnp.reshape(jnp.transpose(token_ids.astype(jnp.int32)), (-1,))
  out_flat = _build(T, S, V, D)(tok_sm, embeddings)
  out5 = jnp.reshape(out_flat, (S, D // 8, T // 128, 8, 128))
  out = jnp.reshape(jnp.transpose(out5, (2, 4, 0, 1, 3)), (T, S, D))
  return out
